# baseline (speedup 1.0000x reference)
import functools

import jax
import jax.numpy as jnp
from jax import lax
from jax.experimental import pallas as pl
from jax.experimental.pallas import tpu as pltpu
from jax.experimental.pallas import tpu_sc as plsc

_INFO = plsc.get_sparse_core_info()
_NC = _INFO.num_cores
_NS = _INFO.num_subcores
_NW = _NC * _NS

_CH = 32


def kernel(X, past, wte, wpe):
    B, S = X.shape
    V, D = wte.shape
    past_len = past.shape[-2]
    n_rows = B * S
    SB = S // _NW
    NSUB = SB // _CH
    NCHUNK = B * NSUB

    X_flat = X.reshape(n_rows).astype(jnp.int32)

    mesh = plsc.VectorSubcoreMesh(core_axis_name="c", subcore_axis_name="s")

    @functools.partial(
        pl.kernel,
        out_type=jax.ShapeDtypeStruct((n_rows, D), jnp.float32),
        mesh=mesh,
        scratch_types=[
            pltpu.VMEM((B, SB), jnp.int32),
            pltpu.VMEM((SB, D), jnp.float32),
            pltpu.VMEM((3, _CH, D), jnp.float32),
            pltpu.SemaphoreType.DMA,
            pltpu.SemaphoreType.DMA,
            pltpu.SemaphoreType.DMA,
            pltpu.SemaphoreType.DMA,
            pltpu.SemaphoreType.DMA,
            pltpu.SemaphoreType.DMA,
        ],
    )
    def emb(wte_hbm, wpe_hbm, xf_hbm, out_hbm, xi, bufw, bufa,
            sg0, sg1, sg2, sw0, sw1, sw2):
        sgs = [sg0, sg1, sg2]
        sws = [sw0, sw1, sw2]
        wid = lax.axis_index("s") * _NC + lax.axis_index("c")
        s0 = wid * SB
        for b in range(B):
            pltpu.sync_copy(xf_hbm.at[pl.ds(b * S + s0, SB)], xi.at[b])
        pltpu.sync_copy(wpe_hbm.at[pl.ds(past_len + s0, SB)], bufw)

        def issue_gather(c):
            slot = c % 3
            b, sub = divmod(c, NSUB)
            return pltpu.async_copy(
                wte_hbm.at[xi.at[b, pl.ds(sub * _CH, _CH)]],
                bufa.at[slot], sgs[slot])

        def issue_writeout(c):
            slot = c % 3
            b, sub = divmod(c, NSUB)
            row0 = b * S + s0 + sub * _CH
            return pltpu.async_copy(
                bufa.at[slot], out_hbm.at[pl.ds(row0, _CH)], sws[slot])

        gathers = [None] * NCHUNK
        wos = [None] * NCHUNK
        gathers[0] = issue_gather(0)
        for c in range(NCHUNK):
            slot = c % 3
            if c + 1 < NCHUNK:
                if c - 2 >= 0:
                    wos[c - 2].wait()
                gathers[c + 1] = issue_gather(c + 1)
            gathers[c].wait()
            sub = c % NSUB
            asl = bufa.at[slot]

            def add_row(r, _):
                for k in range(D // 16):
                    sl = pl.ds(k * 16, 16)
                    plsc.addupdate(asl.at[r, sl], bufw[sub * _CH + r, sl])
                return ()

            lax.fori_loop(0, _CH, add_row, (), unroll=4)
            wos[c] = issue_writeout(c)
        wos[NCHUNK - 2].wait()
        wos[NCHUNK - 1].wait()

    out = emb(wte, wpe, X_flat)
    return out.reshape(B, S, D)


# speedup vs baseline: 1.5716x; 1.5716x over previous
import functools

import jax
import jax.numpy as jnp
from jax import lax
from jax.experimental import pallas as pl
from jax.experimental.pallas import tpu as pltpu
from jax.experimental.pallas import tpu_sc as plsc

_INFO = plsc.get_sparse_core_info()
_NC = _INFO.num_cores
_NS = _INFO.num_subcores
_NW = _NC * _NS

_CH = 32


def kernel(X, past, wte, wpe):
    B, S = X.shape
    V, D = wte.shape
    past_len = past.shape[-2]
    n_rows = B * S
    SB = S // _NW
    NSUB = SB // _CH
    NCHUNK = B * NSUB

    X_flat = X.reshape(n_rows).astype(jnp.int32)

    mesh = plsc.VectorSubcoreMesh(core_axis_name="c", subcore_axis_name="s")

    @functools.partial(
        pl.kernel,
        out_type=jax.ShapeDtypeStruct((n_rows, D), jnp.float32),
        mesh=mesh,
        scratch_types=[
            pltpu.VMEM((B, SB), jnp.int32),
            pltpu.VMEM((SB, D), jnp.float32),
            pltpu.VMEM((3, _CH, D), jnp.float32),
            pltpu.SemaphoreType.DMA,
            pltpu.SemaphoreType.DMA,
            pltpu.SemaphoreType.DMA,
            pltpu.SemaphoreType.DMA,
            pltpu.SemaphoreType.DMA,
            pltpu.SemaphoreType.DMA,
        ],
    )
    def emb(wte_hbm, wpe_hbm, xf_hbm, out_hbm, xi, bufw, bufa,
            sg0, sg1, sg2, sw0, sw1, sw2):
        sgs = [sg0, sg1, sg2]
        sws = [sw0, sw1, sw2]
        wid = lax.axis_index("s") * _NC + lax.axis_index("c")
        s0 = wid * SB
        for b in range(B):
            pltpu.sync_copy(xf_hbm.at[pl.ds(b * S + s0, SB)], xi.at[b])
        pltpu.sync_copy(wpe_hbm.at[pl.ds(past_len + s0, SB)], bufw)

        def issue_gather(c):
            slot = c % 3
            b, sub = divmod(c, NSUB)
            return pltpu.async_copy(
                wte_hbm.at[xi.at[b, pl.ds(sub * _CH, _CH)]],
                bufa.at[slot], sgs[slot])

        def issue_writeout(c):
            slot = c % 3
            b, sub = divmod(c, NSUB)
            row0 = b * S + s0 + sub * _CH
            return pltpu.async_copy(
                bufa.at[slot], out_hbm.at[pl.ds(row0, _CH)], sws[slot])

        gathers = [None] * NCHUNK
        wos = [None] * NCHUNK
        gathers[0] = issue_gather(0)
        for c in range(NCHUNK):
            slot = c % 3
            if c + 1 < NCHUNK:
                if c - 2 >= 0:
                    wos[c - 2].wait()
                gathers[c + 1] = issue_gather(c + 1)
            gathers[c].wait()
            sub = c % NSUB
            asl = bufa.at[slot]

            @functools.partial(plsc.parallel_loop, 0, _CH)
            def add_row(r):
                for k in range(D // 16):
                    sl = pl.ds(k * 16, 16)
                    plsc.addupdate(asl.at[r, sl], bufw[sub * _CH + r, sl])
            wos[c] = issue_writeout(c)
        wos[NCHUNK - 2].wait()
        wos[NCHUNK - 1].wait()

    out = emb(wte, wpe, X_flat)
    return out.reshape(B, S, D)
